# Initial kernel scaffold; baseline (speedup 1.0000x reference)
import functools

import jax
import jax.numpy as jnp
from jax import lax
from jax.experimental import pallas as pl
from jax.experimental.pallas import tpu as pltpu
from jax.experimental.pallas import tpu_sc as plsc

N_NODES = 10000
D = 128
N_EDGES = 320000

NC = 2
NS = 16
NW = NC * NS
CHUNK = 128
NCH = 80
EDGES_PER_TILE = NCH * CHUNK
E_PAD = NW * EDGES_PER_TILE
ACC_ROWS = 10112
ZROWS = 79
ROWS_PER_TILE_OUT = N_NODES // NS
HB = 16

_vector_mesh = plsc.VectorSubcoreMesh(core_axis_name="c", subcore_axis_name="s")


def _tile_ids():
    cid = lax.axis_index("c")
    sid = lax.axis_index("s")
    return cid, sid, cid * NS + sid


@functools.partial(
    pl.kernel,
    out_type=jax.ShapeDtypeStruct((NC, N_NODES, HB), jnp.float32),
    mesh=_vector_mesh,
    scratch_types=[
        pltpu.VMEM((NCH, CHUNK), jnp.int32),
        pltpu.VMEM((CHUNK, HB), jnp.float32),
        pltpu.VMEM((ZROWS, HB), jnp.float32),
        pltpu.VMEM_SHARED((ACC_ROWS, HB), jnp.float32),
    ],
)
def _degree_kernel(dst_hbm, hist_hbm, dst_v, ones_v, z_v, hist_sh):
    cid, sid, wid = _tile_ids()

    @pl.loop(0, CHUNK)
    def _(r):
        ones_v[r, :] = jnp.ones((HB,), jnp.float32)

    @pl.loop(0, ZROWS)
    def _(r):
        z_v[r, :] = jnp.zeros((HB,), jnp.float32)

    @pl.loop(0, 8)
    def _(j):
        pltpu.sync_copy(z_v, hist_sh.at[pl.ds(sid * 632 + j * ZROWS, ZROWS)])

    pltpu.sync_copy(dst_hbm.at[wid], dst_v)
    plsc.subcore_barrier()

    @pl.loop(0, NCH)
    def _(c):
        pltpu.sync_copy(ones_v, hist_sh.at[dst_v.at[c]], add=True)

    plsc.subcore_barrier()
    pltpu.sync_copy(
        hist_sh.at[pl.ds(sid * ROWS_PER_TILE_OUT, ROWS_PER_TILE_OUT)],
        hist_hbm.at[cid, pl.ds(sid * ROWS_PER_TILE_OUT, ROWS_PER_TILE_OUT)],
    )


@functools.partial(
    pl.kernel,
    out_type=jax.ShapeDtypeStruct((NC, N_NODES, D), jnp.float32),
    mesh=_vector_mesh,
    scratch_types=[
        pltpu.VMEM((NCH, CHUNK), jnp.int32),
        pltpu.VMEM((NCH, CHUNK), jnp.int32),
        pltpu.VMEM((CHUNK, D), jnp.float32),
        pltpu.VMEM((CHUNK, D), jnp.float32),
        pltpu.VMEM((ZROWS, D), jnp.float32),
        pltpu.VMEM_SHARED((ACC_ROWS, D), jnp.float32),
        pltpu.SemaphoreType.DMA,
        pltpu.SemaphoreType.DMA,
    ],
)
def _scatter_kernel(table_hbm, src_hbm, dst_hbm, acc_hbm,
                    src_v, dst_v, buf0, buf1, z_v, acc_sh, sem0, sem1):
    cid, sid, wid = _tile_ids()

    @pl.loop(0, ZROWS)
    def _(r):
        @pl.loop(0, D // 16)
        def _(g):
            z_v[r, pl.ds(g * 16, 16)] = jnp.zeros((16,), jnp.float32)

    @pl.loop(0, 8)
    def _(j):
        pltpu.sync_copy(z_v, acc_sh.at[pl.ds(sid * 632 + j * ZROWS, ZROWS)])

    pltpu.sync_copy(src_hbm.at[wid], src_v)
    pltpu.sync_copy(dst_hbm.at[wid], dst_v)
    plsc.subcore_barrier()

    def start(c, buf, sem):
        pltpu.async_copy(table_hbm.at[src_v.at[c]], buf, sem)

    def wait(c, buf, sem):
        pltpu.make_async_copy(table_hbm.at[src_v.at[c]], buf, sem).wait()

    def scat(c, buf):
        pltpu.sync_copy(buf, acc_sh.at[dst_v.at[c]], add=True)

    start(0, buf0, sem0)
    start(1, buf1, sem1)

    @pl.loop(0, NCH - 2, step=2)
    def _(c):
        wait(c, buf0, sem0)
        scat(c, buf0)
        start(c + 2, buf0, sem0)
        wait(c + 1, buf1, sem1)
        scat(c + 1, buf1)
        start(c + 3, buf1, sem1)

    wait(NCH - 2, buf0, sem0)
    scat(NCH - 2, buf0)
    wait(NCH - 1, buf1, sem1)
    scat(NCH - 1, buf1)

    plsc.subcore_barrier()
    pltpu.sync_copy(
        acc_sh.at[pl.ds(sid * ROWS_PER_TILE_OUT, ROWS_PER_TILE_OUT)],
        acc_hbm.at[cid, pl.ds(sid * ROWS_PER_TILE_OUT, ROWS_PER_TILE_OUT)],
    )


BN = 1000


def _erf(x):
    a1, a2, a3, a4, a5 = (0.254829592, -0.284496736, 1.421413741,
                          -1.453152027, 1.061405429)
    p = 0.3275911
    s = jnp.sign(x)
    ax = jnp.abs(x)
    t = 1.0 / (1.0 + p * ax)
    poly = ((((a5 * t + a4) * t + a3) * t + a2) * t + a1) * t
    return s * (1.0 - poly * jnp.exp(-ax * ax))


def _gelu(x):
    return 0.5 * x * (1.0 + _erf(x * 0.7071067811865476))


def _pre_body(x_ref, w_ref, h0_ref, h1_ref, hp_ref, dis_ref):
    deg = h0_ref[:, 0:1] + h1_ref[:, 0:1] + 1.0
    dis = lax.rsqrt(deg)
    h = jnp.dot(x_ref[...], w_ref[...], preferred_element_type=jnp.float32,
                precision=lax.Precision.HIGHEST)
    hp_ref[...] = dis * h
    dis_ref[...] = dis


def _mid_body(a0_ref, a1_ref, hp_ref, dis_ref, b_ref, w_ref, out_ref):
    dis = dis_ref[...]
    s = dis * (a0_ref[...] + a1_ref[...] + hp_ref[...]) + b_ref[...]
    z = _gelu(s)
    out_ref[...] = dis * jnp.dot(z, w_ref[...],
                                 preferred_element_type=jnp.float32,
                                 precision=lax.Precision.HIGHEST)


def _fin_body(a0_ref, a1_ref, hp_ref, dis_ref, b_ref, out_ref):
    out_ref[...] = (dis_ref[...] * (a0_ref[...] + a1_ref[...] + hp_ref[...])
                    + b_ref[...])


_row_spec = pl.BlockSpec((BN, D), lambda i: (i, 0))
_dis_spec = pl.BlockSpec((BN, 1), lambda i: (i, 0))
_hist_spec = pl.BlockSpec((BN, HB), lambda i: (i, 0))
_w_spec = pl.BlockSpec((D, D), lambda i: (0, 0))
_b_spec = pl.BlockSpec((1, D), lambda i: (0, 0))
_grid = (N_NODES // BN,)

_pre_call = pl.pallas_call(
    _pre_body,
    grid=_grid,
    in_specs=[_row_spec, _w_spec, _hist_spec, _hist_spec],
    out_specs=[_row_spec, _dis_spec],
    out_shape=[jax.ShapeDtypeStruct((N_NODES, D), jnp.float32),
               jax.ShapeDtypeStruct((N_NODES, 1), jnp.float32)],
)

_mid_call = pl.pallas_call(
    _mid_body,
    grid=_grid,
    in_specs=[_row_spec, _row_spec, _row_spec, _dis_spec, _b_spec, _w_spec],
    out_specs=_row_spec,
    out_shape=jax.ShapeDtypeStruct((N_NODES, D), jnp.float32),
)

_fin_call = pl.pallas_call(
    _fin_body,
    grid=_grid,
    in_specs=[_row_spec, _row_spec, _row_spec, _dis_spec, _b_spec],
    out_specs=_row_spec,
    out_shape=jax.ShapeDtypeStruct((N_NODES, D), jnp.float32),
)


@jax.jit
def kernel(x_piece, edge_index_piece, W1, b1, W2, b2, W3, b3):
    src = edge_index_piece[0].astype(jnp.int32)
    dst = edge_index_piece[1].astype(jnp.int32)
    npad = E_PAD - N_EDGES
    src_p = jnp.concatenate([src, jnp.zeros((npad,), jnp.int32)])
    dst_p = jnp.concatenate([dst, jnp.full((npad,), N_NODES, jnp.int32)])
    src3 = src_p.reshape(NW, NCH, CHUNK)
    dst3 = dst_p.reshape(NW, NCH, CHUNK)

    hist = _degree_kernel(dst3)
    b1r = b1.reshape(1, D)
    b2r = b2.reshape(1, D)
    b3r = b3.reshape(1, D)

    h1, dis = _pre_call(x_piece, W1, hist[0], hist[1])
    acc = _scatter_kernel(h1, src3, dst3)
    h2 = _mid_call(acc[0], acc[1], h1, dis, b1r, W2)
    acc = _scatter_kernel(h2, src3, dst3)
    h3 = _mid_call(acc[0], acc[1], h2, dis, b2r, W3)
    acc = _scatter_kernel(h3, src3, dst3)
    return _fin_call(acc[0], acc[1], h3, dis, b3r)


# speedup vs baseline: 8.0105x; 8.0105x over previous
import functools

import jax
import jax.numpy as jnp
from jax import lax
from jax.experimental import pallas as pl
from jax.experimental.pallas import tpu as pltpu
from jax.experimental.pallas import tpu_sc as plsc

N_NODES = 10000
D = 128
N_EDGES = 320000

NC = 2
NS = 16
NW = NC * NS
CHUNK = 128
NCH = 80
EDGES_PER_TILE = NCH * CHUNK
E_PAD = NW * EDGES_PER_TILE
ACC_ROWS = 10240
ZROWS = 128
ROWS_PER_TILE = ACC_ROWS // NS
HB = 128

_vector_mesh = plsc.VectorSubcoreMesh(core_axis_name="c", subcore_axis_name="s")


def _tile_ids():
    cid = lax.axis_index("c")
    sid = lax.axis_index("s")
    return cid, sid, cid * NS + sid


@functools.partial(
    pl.kernel,
    out_type=jax.ShapeDtypeStruct((NC, ACC_ROWS, HB), jnp.float32),
    mesh=_vector_mesh,
    scratch_types=[
        pltpu.VMEM((NCH, CHUNK), jnp.int32),
        pltpu.VMEM((CHUNK, HB), jnp.float32),
        pltpu.VMEM_SHARED((ACC_ROWS, HB), jnp.float32),
    ],
)
def _degree_kernel(dst_hbm, hist_hbm, dst_v, ones_v, hist_sh):
    cid, sid, wid = _tile_ids()

    @pl.loop(0, CHUNK)
    def _(r):
        @pl.loop(0, HB // 16)
        def _(g):
            ones_v[r, pl.ds(g * 16, 16)] = jnp.zeros((16,), jnp.float32)

    @pl.loop(0, 5)
    def _(j):
        pltpu.sync_copy(ones_v, hist_sh.at[pl.ds(sid * ROWS_PER_TILE + j * CHUNK, CHUNK)])

    @pl.loop(0, CHUNK)
    def _(r):
        @pl.loop(0, HB // 16)
        def _(g):
            ones_v[r, pl.ds(g * 16, 16)] = jnp.ones((16,), jnp.float32)

    pltpu.sync_copy(dst_hbm.at[wid], dst_v)
    plsc.subcore_barrier()

    @pl.loop(0, NCH)
    def _(c):
        pltpu.sync_copy(ones_v, hist_sh.at[dst_v.at[c]], add=True)

    plsc.subcore_barrier()
    pltpu.sync_copy(
        hist_sh.at[pl.ds(sid * ROWS_PER_TILE, ROWS_PER_TILE)],
        hist_hbm.at[cid, pl.ds(sid * ROWS_PER_TILE, ROWS_PER_TILE)],
    )


NCHH = NCH // 2


@functools.partial(
    pl.kernel,
    out_type=jax.ShapeDtypeStruct((NC, ACC_ROWS, D), jnp.float32),
    mesh=_vector_mesh,
    scratch_types=[
        pltpu.VMEM((NCHH, CHUNK), jnp.int32),
        pltpu.VMEM((NCHH, CHUNK), jnp.int32),
        pltpu.VMEM((CHUNK, D), jnp.float32),
        pltpu.VMEM((CHUNK, D), jnp.float32),
        pltpu.VMEM_SHARED((ACC_ROWS, D), jnp.float32),
        pltpu.SemaphoreType.DMA,
        pltpu.SemaphoreType.DMA,
    ],
)
def _scatter_kernel(table_hbm, src_hbm, dst_hbm, acc_hbm,
                    src_v, dst_v, buf0, buf1, acc_sh, sem0, sem1):
    cid, sid, wid = _tile_ids()

    @pl.loop(0, CHUNK)
    def _(r):
        @pl.loop(0, D // 16)
        def _(g):
            buf0[r, pl.ds(g * 16, 16)] = jnp.zeros((16,), jnp.float32)

    @pl.loop(0, 5)
    def _(j):
        pltpu.sync_copy(buf0, acc_sh.at[pl.ds(sid * ROWS_PER_TILE + j * CHUNK, CHUNK)])

    plsc.subcore_barrier()

    def start(c, buf, sem):
        pltpu.async_copy(table_hbm.at[src_v.at[c]], buf, sem)

    def wait(c, buf, sem):
        pltpu.make_async_copy(table_hbm.at[src_v.at[c]], buf, sem).wait()

    def scat(c, buf):
        pltpu.sync_copy(buf, acc_sh.at[dst_v.at[c]], add=True)

    for p in range(2):
        pltpu.sync_copy(src_hbm.at[wid, pl.ds(p * NCHH, NCHH)], src_v)
        pltpu.sync_copy(dst_hbm.at[wid, pl.ds(p * NCHH, NCHH)], dst_v)
        start(0, buf0, sem0)
        start(1, buf1, sem1)

        @pl.loop(0, NCHH - 2, step=2)
        def _(c):
            wait(c, buf0, sem0)
            scat(c, buf0)
            start(c + 2, buf0, sem0)
            wait(c + 1, buf1, sem1)
            scat(c + 1, buf1)
            start(c + 3, buf1, sem1)

        wait(NCHH - 2, buf0, sem0)
        scat(NCHH - 2, buf0)
        wait(NCHH - 1, buf1, sem1)
        scat(NCHH - 1, buf1)

    plsc.subcore_barrier()
    pltpu.sync_copy(
        acc_sh.at[pl.ds(sid * ROWS_PER_TILE, ROWS_PER_TILE)],
        acc_hbm.at[cid, pl.ds(sid * ROWS_PER_TILE, ROWS_PER_TILE)],
    )


BN = 1000


def _erf(x):
    a1, a2, a3, a4, a5 = (0.254829592, -0.284496736, 1.421413741,
                          -1.453152027, 1.061405429)
    p = 0.3275911
    s = jnp.sign(x)
    ax = jnp.abs(x)
    t = 1.0 / (1.0 + p * ax)
    poly = ((((a5 * t + a4) * t + a3) * t + a2) * t + a1) * t
    return s * (1.0 - poly * jnp.exp(-ax * ax))


def _gelu(x):
    return 0.5 * x * (1.0 + _erf(x * 0.7071067811865476))


def _pre_body(x_ref, w_ref, h0_ref, h1_ref, hp_ref, dis_ref):
    deg = h0_ref[:, 0:1] + h1_ref[:, 0:1] + 1.0
    dis = lax.rsqrt(deg)
    h = jnp.dot(x_ref[...], w_ref[...], preferred_element_type=jnp.float32,
                precision=lax.Precision.HIGHEST)
    hp_ref[...] = dis * h
    dis_ref[...] = dis


def _mid_body(a0_ref, a1_ref, hp_ref, dis_ref, b_ref, w_ref, out_ref):
    dis = dis_ref[...]
    s = dis * (a0_ref[...] + a1_ref[...] + hp_ref[...]) + b_ref[...]
    z = _gelu(s)
    out_ref[...] = dis * jnp.dot(z, w_ref[...],
                                 preferred_element_type=jnp.float32,
                                 precision=lax.Precision.HIGHEST)


def _fin_body(a0_ref, a1_ref, hp_ref, dis_ref, b_ref, out_ref):
    out_ref[...] = (dis_ref[...] * (a0_ref[...] + a1_ref[...] + hp_ref[...])
                    + b_ref[...])


_row_spec = pl.BlockSpec((BN, D), lambda i: (i, 0))
_dis_spec = pl.BlockSpec((BN, 1), lambda i: (i, 0))
_hist_spec = pl.BlockSpec((BN, HB), lambda i: (i, 0))
_w_spec = pl.BlockSpec((D, D), lambda i: (0, 0))
_b_spec = pl.BlockSpec((1, D), lambda i: (0, 0))
_grid = (N_NODES // BN,)

_pre_call = pl.pallas_call(
    _pre_body,
    grid=_grid,
    in_specs=[_row_spec, _w_spec, _hist_spec, _hist_spec],
    out_specs=[_row_spec, _dis_spec],
    out_shape=[jax.ShapeDtypeStruct((N_NODES, D), jnp.float32),
               jax.ShapeDtypeStruct((N_NODES, 1), jnp.float32)],
)

_mid_call = pl.pallas_call(
    _mid_body,
    grid=_grid,
    in_specs=[_row_spec, _row_spec, _row_spec, _dis_spec, _b_spec, _w_spec],
    out_specs=_row_spec,
    out_shape=jax.ShapeDtypeStruct((N_NODES, D), jnp.float32),
)

_fin_call = pl.pallas_call(
    _fin_body,
    grid=_grid,
    in_specs=[_row_spec, _row_spec, _row_spec, _dis_spec, _b_spec],
    out_specs=_row_spec,
    out_shape=jax.ShapeDtypeStruct((N_NODES, D), jnp.float32),
)


@jax.jit
def kernel(x_piece, edge_index_piece, W1, b1, W2, b2, W3, b3):
    src = edge_index_piece[0].astype(jnp.int32)
    dst = edge_index_piece[1].astype(jnp.int32)
    npad = E_PAD - N_EDGES
    src_p = jnp.concatenate([src, jnp.zeros((npad,), jnp.int32)])
    dst_p = jnp.concatenate([dst, jnp.full((npad,), N_NODES, jnp.int32)])
    src3 = src_p.reshape(NW, NCH, CHUNK)
    dst3 = dst_p.reshape(NW, NCH, CHUNK)

    hist_p = _degree_kernel(dst3)
    hist = hist_p[:, :N_NODES]
    b1r = b1.reshape(1, D)
    b2r = b2.reshape(1, D)
    b3r = b3.reshape(1, D)

    h1, dis = _pre_call(x_piece, W1, hist[0], hist[1])
    acc = _scatter_kernel(h1, src3, dst3)[:, :N_NODES]
    h2 = _mid_call(acc[0], acc[1], h1, dis, b1r, W2)
    acc = _scatter_kernel(h2, src3, dst3)[:, :N_NODES]
    h3 = _mid_call(acc[0], acc[1], h2, dis, b2r, W3)
    acc = _scatter_kernel(h3, src3, dst3)[:, :N_NODES]
    return _fin_call(acc[0], acc[1], h3, dis, b3r)


# speedup vs baseline: 23.5780x; 2.9434x over previous
import functools

import jax
import jax.numpy as jnp
from jax import lax
from jax.experimental import pallas as pl
from jax.experimental.pallas import tpu as pltpu
from jax.experimental.pallas import tpu_sc as plsc

N_NODES = 10000
D = 128
N_EDGES = 320000

NC = 2
NS = 16
NW = NC * NS
CHUNK = 128
NCH = 80
EDGES_PER_TILE = NCH * CHUNK
E_PAD = NW * EDGES_PER_TILE
ACC_ROWS = 10240
ZROWS = 128
ROWS_PER_TILE = ACC_ROWS // NS
HB = 128

_vector_mesh = plsc.VectorSubcoreMesh(core_axis_name="c", subcore_axis_name="s")


def _tile_ids():
    cid = lax.axis_index("c")
    sid = lax.axis_index("s")
    return cid, sid, cid * NS + sid


@functools.partial(
    pl.kernel,
    out_type=jax.ShapeDtypeStruct((NC, ACC_ROWS, HB), jnp.float32),
    mesh=_vector_mesh,
    scratch_types=[
        pltpu.VMEM((NCH, CHUNK), jnp.int32),
        pltpu.VMEM((CHUNK, HB), jnp.float32),
        pltpu.VMEM_SHARED((ACC_ROWS, HB), jnp.float32),
    ],
)
def _degree_kernel(dst_hbm, hist_hbm, dst_v, ones_v, hist_sh):
    cid, sid, wid = _tile_ids()

    @pl.loop(0, CHUNK)
    def _(r):
        @pl.loop(0, HB // 16)
        def _(g):
            ones_v[r, pl.ds(g * 16, 16)] = jnp.zeros((16,), jnp.float32)

    @pl.loop(0, 5)
    def _(j):
        pltpu.sync_copy(ones_v, hist_sh.at[pl.ds(sid * ROWS_PER_TILE + j * CHUNK, CHUNK)])

    @pl.loop(0, CHUNK)
    def _(r):
        @pl.loop(0, HB // 16)
        def _(g):
            ones_v[r, pl.ds(g * 16, 16)] = jnp.ones((16,), jnp.float32)

    pltpu.sync_copy(dst_hbm.at[wid], dst_v)
    plsc.subcore_barrier()

    @pl.loop(0, NCH)
    def _(c):
        pltpu.sync_copy(ones_v, hist_sh.at[dst_v.at[c]], add=True)

    plsc.subcore_barrier()
    pltpu.sync_copy(
        hist_sh.at[pl.ds(sid * ROWS_PER_TILE, ROWS_PER_TILE)],
        hist_hbm.at[cid, pl.ds(sid * ROWS_PER_TILE, ROWS_PER_TILE)],
    )


NCHH = NCH // 2


@functools.partial(
    pl.kernel,
    out_type=jax.ShapeDtypeStruct((NC, ACC_ROWS, D), jnp.float32),
    mesh=_vector_mesh,
    scratch_types=[
        pltpu.VMEM((NCHH, CHUNK), jnp.int32),
        pltpu.VMEM((NCHH, CHUNK), jnp.int32),
        pltpu.VMEM((CHUNK, D), jnp.float32),
        pltpu.VMEM((CHUNK, D), jnp.float32),
        pltpu.VMEM_SHARED((ACC_ROWS, D), jnp.float32),
        pltpu.SemaphoreType.DMA,
        pltpu.SemaphoreType.DMA,
    ],
)
def _scatter_kernel(table_hbm, src_hbm, dst_hbm, acc_hbm,
                    src_v, dst_v, buf0, buf1, acc_sh, sem0, sem1):
    cid, sid, wid = _tile_ids()

    @pl.loop(0, CHUNK)
    def _(r):
        @pl.loop(0, D // 16)
        def _(g):
            buf0[r, pl.ds(g * 16, 16)] = jnp.zeros((16,), jnp.float32)

    @pl.loop(0, 5)
    def _(j):
        pltpu.sync_copy(buf0, acc_sh.at[pl.ds(sid * ROWS_PER_TILE + j * CHUNK, CHUNK)])

    plsc.subcore_barrier()

    def start(c, buf, sem):
        pltpu.async_copy(table_hbm.at[src_v.at[c]], buf, sem)

    def wait(c, buf, sem):
        pltpu.make_async_copy(table_hbm.at[src_v.at[c]], buf, sem).wait()

    def scat(c, buf):
        pltpu.sync_copy(buf, acc_sh.at[dst_v.at[c]], add=True)

    for p in range(2):
        pltpu.sync_copy(src_hbm.at[wid, pl.ds(p * NCHH, NCHH)], src_v)
        pltpu.sync_copy(dst_hbm.at[wid, pl.ds(p * NCHH, NCHH)], dst_v)
        start(0, buf0, sem0)
        start(1, buf1, sem1)

        @pl.loop(0, NCHH - 2, step=2)
        def _(c):
            wait(c, buf0, sem0)
            scat(c, buf0)
            start(c + 2, buf0, sem0)
            wait(c + 1, buf1, sem1)
            scat(c + 1, buf1)
            start(c + 3, buf1, sem1)

        wait(NCHH - 2, buf0, sem0)
        scat(NCHH - 2, buf0)
        wait(NCHH - 1, buf1, sem1)
        scat(NCHH - 1, buf1)

    plsc.subcore_barrier()
    pltpu.sync_copy(
        acc_sh.at[pl.ds(sid * ROWS_PER_TILE, ROWS_PER_TILE)],
        acc_hbm.at[cid, pl.ds(sid * ROWS_PER_TILE, ROWS_PER_TILE)],
    )


BN = 1000


def _erf(x):
    a1, a2, a3, a4, a5 = (0.254829592, -0.284496736, 1.421413741,
                          -1.453152027, 1.061405429)
    p = 0.3275911
    s = jnp.sign(x)
    ax = jnp.abs(x)
    t = 1.0 / (1.0 + p * ax)
    poly = ((((a5 * t + a4) * t + a3) * t + a2) * t + a1) * t
    return s * (1.0 - poly * jnp.exp(-ax * ax))


def _gelu(x):
    return 0.5 * x * (1.0 + _erf(x * 0.7071067811865476))


def _pre_body(x_ref, w_ref, h0_ref, h1_ref, hp_ref, dis_ref):
    deg = h0_ref[:, 0:1] + h1_ref[:, 0:1] + 1.0
    dis = lax.rsqrt(deg)
    h = jnp.dot(x_ref[...], w_ref[...], preferred_element_type=jnp.float32,
                precision=lax.Precision.HIGHEST)
    hp_ref[...] = dis * h
    dis_ref[...] = dis


def _mid_body(a0_ref, a1_ref, hp_ref, dis_ref, b_ref, w_ref, out_ref):
    dis = dis_ref[...]
    s = dis * (a0_ref[...] + a1_ref[...] + hp_ref[...]) + b_ref[...]
    z = _gelu(s)
    out_ref[...] = dis * jnp.dot(z, w_ref[...],
                                 preferred_element_type=jnp.float32,
                                 precision=lax.Precision.HIGHEST)


def _fin_body(a0_ref, a1_ref, hp_ref, dis_ref, b_ref, out_ref):
    out_ref[...] = (dis_ref[...] * (a0_ref[...] + a1_ref[...] + hp_ref[...])
                    + b_ref[...])


_row_spec = pl.BlockSpec((BN, D), lambda i: (i, 0))
_dis_spec = pl.BlockSpec((BN, 1), lambda i: (i, 0))
_hist_spec = pl.BlockSpec((BN, HB), lambda i: (i, 0))
_w_spec = pl.BlockSpec((D, D), lambda i: (0, 0))
_b_spec = pl.BlockSpec((1, D), lambda i: (0, 0))
_grid = (N_NODES // BN,)

_pre_call = pl.pallas_call(
    _pre_body,
    grid=_grid,
    in_specs=[_row_spec, _w_spec, _hist_spec, _hist_spec],
    out_specs=[_row_spec, _dis_spec],
    out_shape=[jax.ShapeDtypeStruct((N_NODES, D), jnp.float32),
               jax.ShapeDtypeStruct((N_NODES, 1), jnp.float32)],
)

_mid_call = pl.pallas_call(
    _mid_body,
    grid=_grid,
    in_specs=[_row_spec, _row_spec, _row_spec, _dis_spec, _b_spec, _w_spec],
    out_specs=_row_spec,
    out_shape=jax.ShapeDtypeStruct((N_NODES, D), jnp.float32),
)

_fin_call = pl.pallas_call(
    _fin_body,
    grid=_grid,
    in_specs=[_row_spec, _row_spec, _row_spec, _dis_spec, _b_spec],
    out_specs=_row_spec,
    out_shape=jax.ShapeDtypeStruct((N_NODES, D), jnp.float32),
)


@jax.jit
def kernel(x_piece, edge_index_piece, W1, b1, W2, b2, W3, b3):
    src = edge_index_piece[0].astype(jnp.int32)
    dst = edge_index_piece[1].astype(jnp.int32)
    npad = E_PAD - N_EDGES
    pad_i = jnp.arange(npad, dtype=jnp.int32)
    src_p = jnp.concatenate([src, pad_i % N_NODES])
    dst_p = jnp.concatenate([dst, N_NODES + pad_i % (ACC_ROWS - N_NODES)])
    src3 = src_p.reshape(NW, NCH, CHUNK)
    dst3 = dst_p.reshape(NW, NCH, CHUNK)

    hist_p = _degree_kernel(dst3)
    hist = hist_p[:, :N_NODES]
    b1r = b1.reshape(1, D)
    b2r = b2.reshape(1, D)
    b3r = b3.reshape(1, D)

    h1, dis = _pre_call(x_piece, W1, hist[0], hist[1])
    acc = _scatter_kernel(h1, src3, dst3)[:, :N_NODES]
    h2 = _mid_call(acc[0], acc[1], h1, dis, b1r, W2)
    acc = _scatter_kernel(h2, src3, dst3)[:, :N_NODES]
    h3 = _mid_call(acc[0], acc[1], h2, dis, b2r, W3)
    acc = _scatter_kernel(h3, src3, dst3)[:, :N_NODES]
    return _fin_call(acc[0], acc[1], h3, dis, b3r)


# speedup vs baseline: 25.1082x; 1.0649x over previous
import functools

import jax
import jax.numpy as jnp
from jax import lax
from jax.experimental import pallas as pl
from jax.experimental.pallas import tpu as pltpu
from jax.experimental.pallas import tpu_sc as plsc

N_NODES = 10000
D = 128
N_EDGES = 320000

NC = 2
NS = 16
NW = NC * NS
CHUNK = 128
NCH = 80
EDGES_PER_TILE = NCH * CHUNK
E_PAD = NW * EDGES_PER_TILE
ACC_ROWS = 10240
ZROWS = 128
ROWS_PER_TILE = ACC_ROWS // NS
HB = 128

_vector_mesh = plsc.VectorSubcoreMesh(core_axis_name="c", subcore_axis_name="s")


def _tile_ids():
    cid = lax.axis_index("c")
    sid = lax.axis_index("s")
    return cid, sid, cid * NS + sid


@functools.partial(
    pl.kernel,
    out_type=jax.ShapeDtypeStruct((NC, ACC_ROWS, HB), jnp.float32),
    mesh=_vector_mesh,
    scratch_types=[
        pltpu.VMEM((NCH, CHUNK), jnp.int32),
        pltpu.VMEM((CHUNK, HB), jnp.float32),
        pltpu.VMEM_SHARED((ACC_ROWS, HB), jnp.float32),
    ],
)
def _degree_kernel(dst_hbm, hist_hbm, dst_v, ones_v, hist_sh):
    cid, sid, wid = _tile_ids()

    @pl.loop(0, CHUNK)
    def _(r):
        @pl.loop(0, HB // 16)
        def _(g):
            ones_v[r, pl.ds(g * 16, 16)] = jnp.zeros((16,), jnp.float32)

    @pl.loop(0, 5)
    def _(j):
        pltpu.sync_copy(ones_v, hist_sh.at[pl.ds(sid * ROWS_PER_TILE + j * CHUNK, CHUNK)])

    @pl.loop(0, CHUNK)
    def _(r):
        @pl.loop(0, HB // 16)
        def _(g):
            ones_v[r, pl.ds(g * 16, 16)] = jnp.ones((16,), jnp.float32)

    pltpu.sync_copy(dst_hbm.at[wid], dst_v)
    plsc.subcore_barrier()

    @pl.loop(0, NCH)
    def _(c):
        pltpu.sync_copy(ones_v, hist_sh.at[dst_v.at[c]], add=True)

    plsc.subcore_barrier()
    pltpu.sync_copy(
        hist_sh.at[pl.ds(sid * ROWS_PER_TILE, ROWS_PER_TILE)],
        hist_hbm.at[cid, pl.ds(sid * ROWS_PER_TILE, ROWS_PER_TILE)],
    )


NCHH = NCH // 2


@functools.partial(
    pl.kernel,
    out_type=jax.ShapeDtypeStruct((NC, ACC_ROWS, D), jnp.float32),
    mesh=_vector_mesh,
    scratch_types=[
        pltpu.VMEM((NCHH, CHUNK), jnp.int32),
        pltpu.VMEM((NCHH, CHUNK), jnp.int32),
        pltpu.VMEM((CHUNK, D), jnp.float32),
        pltpu.VMEM((CHUNK, D), jnp.float32),
        pltpu.VMEM_SHARED((ACC_ROWS, D), jnp.float32),
        pltpu.SemaphoreType.DMA,
        pltpu.SemaphoreType.DMA,
    ],
)
def _scatter_kernel(table_hbm, src_hbm, dst_hbm, acc_hbm,
                    src_v, dst_v, buf0, buf1, acc_sh, sem0, sem1):
    cid, sid, wid = _tile_ids()

    @pl.loop(0, CHUNK)
    def _(r):
        @pl.loop(0, D // 16)
        def _(g):
            buf0[r, pl.ds(g * 16, 16)] = jnp.zeros((16,), jnp.float32)

    @pl.loop(0, 5)
    def _(j):
        pltpu.sync_copy(buf0, acc_sh.at[pl.ds(sid * ROWS_PER_TILE + j * CHUNK, CHUNK)])

    plsc.subcore_barrier()

    def start(c, buf, sem):
        pltpu.async_copy(table_hbm.at[src_v.at[c]], buf, sem)

    def wait(c, buf, sem):
        pltpu.make_async_copy(table_hbm.at[src_v.at[c]], buf, sem).wait()

    def scat(c, buf):
        pltpu.sync_copy(buf, acc_sh.at[dst_v.at[c]], add=True)

    for p in range(2):
        pltpu.sync_copy(src_hbm.at[wid, pl.ds(p * NCHH, NCHH)], src_v)
        pltpu.sync_copy(dst_hbm.at[wid, pl.ds(p * NCHH, NCHH)], dst_v)
        start(0, buf0, sem0)
        start(1, buf1, sem1)

        @pl.loop(0, NCHH - 2, step=2)
        def _(c):
            wait(c, buf0, sem0)
            scat(c, buf0)
            start(c + 2, buf0, sem0)
            wait(c + 1, buf1, sem1)
            scat(c + 1, buf1)
            start(c + 3, buf1, sem1)

        wait(NCHH - 2, buf0, sem0)
        scat(NCHH - 2, buf0)
        wait(NCHH - 1, buf1, sem1)
        scat(NCHH - 1, buf1)

    plsc.subcore_barrier()
    pltpu.sync_copy(
        acc_sh.at[pl.ds(sid * ROWS_PER_TILE, ROWS_PER_TILE)],
        acc_hbm.at[cid, pl.ds(sid * ROWS_PER_TILE, ROWS_PER_TILE)],
    )


BN = 1000


def _erf(x):
    a1, a2, a3, a4, a5 = (0.254829592, -0.284496736, 1.421413741,
                          -1.453152027, 1.061405429)
    p = 0.3275911
    s = jnp.sign(x)
    ax = jnp.abs(x)
    t = 1.0 / (1.0 + p * ax)
    poly = ((((a5 * t + a4) * t + a3) * t + a2) * t + a1) * t
    return s * (1.0 - poly * jnp.exp(-ax * ax))


def _gelu(x):
    return 0.5 * x * (1.0 + _erf(x * 0.7071067811865476))


def _mm_body(x_ref, w_ref, h_ref):
    h_ref[...] = jnp.dot(x_ref[...], w_ref[...],
                         preferred_element_type=jnp.float32,
                         precision=lax.Precision.HIGHEST)


def _scale_body(h_ref, g0_ref, g1_ref, hp_ref, dis_ref):
    deg = g0_ref[0, :, 0:1] + g1_ref[0, :, 0:1] + 1.0
    dis = lax.rsqrt(deg)
    hp_ref[...] = dis * h_ref[...]
    dis_ref[...] = dis


def _mid_body(a0_ref, a1_ref, hp_ref, dis_ref, b_ref, w_ref, out_ref):
    dis = dis_ref[...]
    s = dis * (a0_ref[0] + a1_ref[0] + hp_ref[...]) + b_ref[...]
    z = _gelu(s)
    out_ref[...] = dis * jnp.dot(z, w_ref[...],
                                 preferred_element_type=jnp.float32,
                                 precision=lax.Precision.HIGHEST)


def _fin_body(a0_ref, a1_ref, hp_ref, dis_ref, b_ref, out_ref):
    out_ref[...] = (dis_ref[...] * (a0_ref[0] + a1_ref[0] + hp_ref[...])
                    + b_ref[...])


_row_spec = pl.BlockSpec((BN, D), lambda i: (i, 0))
_dis_spec = pl.BlockSpec((BN, 1), lambda i: (i, 0))
_g0_spec = pl.BlockSpec((1, BN, HB), lambda i: (0, i, 0))
_g1_spec = pl.BlockSpec((1, BN, HB), lambda i: (1, i, 0))
_a0_spec = pl.BlockSpec((1, BN, D), lambda i: (0, i, 0))
_a1_spec = pl.BlockSpec((1, BN, D), lambda i: (1, i, 0))
_w_spec = pl.BlockSpec((D, D), lambda i: (0, 0))
_b_spec = pl.BlockSpec((1, D), lambda i: (0, 0))
_grid = (N_NODES // BN,)

_mm_call = pl.pallas_call(
    _mm_body,
    grid=_grid,
    in_specs=[_row_spec, _w_spec],
    out_specs=_row_spec,
    out_shape=jax.ShapeDtypeStruct((N_NODES, D), jnp.float32),
)

_scale_call = pl.pallas_call(
    _scale_body,
    grid=_grid,
    in_specs=[_row_spec, _g0_spec, _g1_spec],
    out_specs=[_row_spec, _dis_spec],
    out_shape=[jax.ShapeDtypeStruct((N_NODES, D), jnp.float32),
               jax.ShapeDtypeStruct((N_NODES, 1), jnp.float32)],
)

_mid_call = pl.pallas_call(
    _mid_body,
    grid=_grid,
    in_specs=[_a0_spec, _a1_spec, _row_spec, _dis_spec, _b_spec, _w_spec],
    out_specs=_row_spec,
    out_shape=jax.ShapeDtypeStruct((N_NODES, D), jnp.float32),
)

_fin_call = pl.pallas_call(
    _fin_body,
    grid=_grid,
    in_specs=[_a0_spec, _a1_spec, _row_spec, _dis_spec, _b_spec],
    out_specs=_row_spec,
    out_shape=jax.ShapeDtypeStruct((N_NODES, D), jnp.float32),
)


@jax.jit
def kernel(x_piece, edge_index_piece, W1, b1, W2, b2, W3, b3):
    src = edge_index_piece[0].astype(jnp.int32)
    dst = edge_index_piece[1].astype(jnp.int32)
    npad = E_PAD - N_EDGES
    pad_i = jnp.arange(npad, dtype=jnp.int32)
    src_p = jnp.concatenate([src, pad_i % N_NODES])
    dst_p = jnp.concatenate([dst, N_NODES + pad_i % (ACC_ROWS - N_NODES)])
    src3 = src_p.reshape(NW, NCH, CHUNK)
    dst3 = dst_p.reshape(NW, NCH, CHUNK)

    b1r = b1.reshape(1, D)
    b2r = b2.reshape(1, D)
    b3r = b3.reshape(1, D)

    hist_p = _degree_kernel(dst3)
    h1_raw = _mm_call(x_piece, W1)
    h1, dis = _scale_call(h1_raw, hist_p, hist_p)
    acc = _scatter_kernel(h1, src3, dst3)
    h2 = _mid_call(acc, acc, h1, dis, b1r, W2)
    acc = _scatter_kernel(h2, src3, dst3)
    h3 = _mid_call(acc, acc, h2, dis, b2r, W3)
    acc = _scatter_kernel(h3, src3, dst3)
    return _fin_call(acc, acc, h3, dis, b3r)


# speedup vs baseline: 25.6792x; 1.0227x over previous
import functools

import jax
import jax.numpy as jnp
from jax import lax
from jax.experimental import pallas as pl
from jax.experimental.pallas import tpu as pltpu
from jax.experimental.pallas import tpu_sc as plsc

N_NODES = 10000
D = 128
N_EDGES = 320000

NC = 2
NS = 16
NW = NC * NS
CHUNK = 128
NCH = 80
EDGES_PER_TILE = NCH * CHUNK
E_PAD = NW * EDGES_PER_TILE
ACC_ROWS = 10240
ZROWS = 128
ROWS_PER_TILE = ACC_ROWS // NS
HB = 128

_vector_mesh = plsc.VectorSubcoreMesh(core_axis_name="c", subcore_axis_name="s")


def _tile_ids():
    cid = lax.axis_index("c")
    sid = lax.axis_index("s")
    return cid, sid, cid * NS + sid


@functools.partial(
    pl.kernel,
    out_type=jax.ShapeDtypeStruct((NC, ACC_ROWS, HB), jnp.float32),
    mesh=_vector_mesh,
    scratch_types=[
        pltpu.VMEM((NCH, CHUNK), jnp.int32),
        pltpu.VMEM((CHUNK, HB), jnp.float32),
        pltpu.VMEM_SHARED((ACC_ROWS, HB), jnp.float32),
    ],
)
def _degree_kernel(dst_hbm, hist_hbm, dst_v, ones_v, hist_sh):
    cid, sid, wid = _tile_ids()

    @pl.loop(0, CHUNK)
    def _(r):
        @pl.loop(0, HB // 16)
        def _(g):
            ones_v[r, pl.ds(g * 16, 16)] = jnp.zeros((16,), jnp.float32)

    @pl.loop(0, 5)
    def _(j):
        pltpu.sync_copy(ones_v, hist_sh.at[pl.ds(sid * ROWS_PER_TILE + j * CHUNK, CHUNK)])

    @pl.loop(0, CHUNK)
    def _(r):
        @pl.loop(0, HB // 16)
        def _(g):
            ones_v[r, pl.ds(g * 16, 16)] = jnp.ones((16,), jnp.float32)

    pltpu.sync_copy(dst_hbm.at[wid], dst_v)
    plsc.subcore_barrier()

    @pl.loop(0, NCH)
    def _(c):
        pltpu.sync_copy(ones_v, hist_sh.at[dst_v.at[c]], add=True)

    plsc.subcore_barrier()
    pltpu.sync_copy(
        hist_sh.at[pl.ds(sid * ROWS_PER_TILE, ROWS_PER_TILE)],
        hist_hbm.at[cid, pl.ds(sid * ROWS_PER_TILE, ROWS_PER_TILE)],
    )


NCHH = NCH // 2


@functools.partial(
    pl.kernel,
    out_type=jax.ShapeDtypeStruct((NC, ACC_ROWS, D), jnp.float32),
    mesh=_vector_mesh,
    scratch_types=[
        pltpu.VMEM((NCHH, CHUNK), jnp.int32),
        pltpu.VMEM((NCHH, CHUNK), jnp.int32),
        pltpu.VMEM((CHUNK, D), jnp.float32),
        pltpu.VMEM((CHUNK, D), jnp.float32),
        pltpu.VMEM_SHARED((ACC_ROWS, D), jnp.float32),
        pltpu.SemaphoreType.DMA,
        pltpu.SemaphoreType.DMA,
    ],
)
def _scatter_kernel(table_hbm, src_hbm, dst_hbm, acc_hbm,
                    src_v, dst_v, buf0, buf1, acc_sh, sem0, sem1):
    cid, sid, wid = _tile_ids()

    @pl.loop(0, CHUNK)
    def _(r):
        @pl.loop(0, D // 16)
        def _(g):
            buf0[r, pl.ds(g * 16, 16)] = jnp.zeros((16,), jnp.float32)

    @pl.loop(0, 5)
    def _(j):
        pltpu.sync_copy(buf0, acc_sh.at[pl.ds(sid * ROWS_PER_TILE + j * CHUNK, CHUNK)])

    plsc.subcore_barrier()

    def start(c, buf, sem):
        pltpu.async_copy(table_hbm.at[src_v.at[c]], buf, sem)

    def wait(c, buf, sem):
        pltpu.make_async_copy(table_hbm.at[src_v.at[c]], buf, sem).wait()

    def scat(c, buf):
        pltpu.sync_copy(buf, acc_sh.at[dst_v.at[c]], add=True)

    for p in range(2):
        pltpu.sync_copy(src_hbm.at[wid, pl.ds(p * NCHH, NCHH)], src_v)
        pltpu.sync_copy(dst_hbm.at[wid, pl.ds(p * NCHH, NCHH)], dst_v)
        start(0, buf0, sem0)
        start(1, buf1, sem1)

        @pl.loop(0, NCHH - 2, step=2)
        def _(c):
            wait(c, buf0, sem0)
            scat(c, buf0)
            start(c + 2, buf0, sem0)
            wait(c + 1, buf1, sem1)
            scat(c + 1, buf1)
            start(c + 3, buf1, sem1)

        wait(NCHH - 2, buf0, sem0)
        scat(NCHH - 2, buf0)
        wait(NCHH - 1, buf1, sem1)
        scat(NCHH - 1, buf1)

    plsc.subcore_barrier()
    pltpu.sync_copy(
        acc_sh.at[pl.ds(sid * ROWS_PER_TILE, ROWS_PER_TILE)],
        acc_hbm.at[cid, pl.ds(sid * ROWS_PER_TILE, ROWS_PER_TILE)],
    )


BN = 2000


def _erf(x):
    a1, a2, a3, a4, a5 = (0.254829592, -0.284496736, 1.421413741,
                          -1.453152027, 1.061405429)
    p = 0.3275911
    s = jnp.sign(x)
    ax = jnp.abs(x)
    t = 1.0 / (1.0 + p * ax)
    poly = ((((a5 * t + a4) * t + a3) * t + a2) * t + a1) * t
    return s * (1.0 - poly * jnp.exp(-ax * ax))


def _gelu(x):
    return 0.5 * x * (1.0 + _erf(x * 0.7071067811865476))


def _mm_body(x_ref, w_ref, h_ref):
    h_ref[...] = jnp.dot(x_ref[...], w_ref[...],
                         preferred_element_type=jnp.float32,
                         precision=lax.Precision.HIGHEST)


def _scale_body(h_ref, g0_ref, g1_ref, hp_ref, dis_ref):
    deg = g0_ref[0, :, 0:1] + g1_ref[0, :, 0:1] + 1.0
    dis = lax.rsqrt(deg)
    hp_ref[...] = dis * h_ref[...]
    dis_ref[...] = dis


def _mid_body(a0_ref, a1_ref, hp_ref, dis_ref, b_ref, w_ref, out_ref):
    dis = dis_ref[...]
    s = dis * (a0_ref[0] + a1_ref[0] + hp_ref[...]) + b_ref[...]
    z = _gelu(s)
    out_ref[...] = dis * jnp.dot(z, w_ref[...],
                                 preferred_element_type=jnp.float32,
                                 precision=lax.Precision.HIGHEST)


def _fin_body(a0_ref, a1_ref, hp_ref, dis_ref, b_ref, out_ref):
    out_ref[...] = (dis_ref[...] * (a0_ref[0] + a1_ref[0] + hp_ref[...])
                    + b_ref[...])


_row_spec = pl.BlockSpec((BN, D), lambda i: (i, 0))
_dis_spec = pl.BlockSpec((BN, 1), lambda i: (i, 0))
_g0_spec = pl.BlockSpec((1, BN, HB), lambda i: (0, i, 0))
_g1_spec = pl.BlockSpec((1, BN, HB), lambda i: (1, i, 0))
_a0_spec = pl.BlockSpec((1, BN, D), lambda i: (0, i, 0))
_a1_spec = pl.BlockSpec((1, BN, D), lambda i: (1, i, 0))
_w_spec = pl.BlockSpec((D, D), lambda i: (0, 0))
_b_spec = pl.BlockSpec((1, D), lambda i: (0, 0))
_grid = (N_NODES // BN,)

_mm_call = pl.pallas_call(
    _mm_body,
    grid=_grid,
    in_specs=[_row_spec, _w_spec],
    out_specs=_row_spec,
    out_shape=jax.ShapeDtypeStruct((N_NODES, D), jnp.float32),
)

_scale_call = pl.pallas_call(
    _scale_body,
    grid=_grid,
    in_specs=[_row_spec, _g0_spec, _g1_spec],
    out_specs=[_row_spec, _dis_spec],
    out_shape=[jax.ShapeDtypeStruct((N_NODES, D), jnp.float32),
               jax.ShapeDtypeStruct((N_NODES, 1), jnp.float32)],
)

_mid_call = pl.pallas_call(
    _mid_body,
    grid=_grid,
    in_specs=[_a0_spec, _a1_spec, _row_spec, _dis_spec, _b_spec, _w_spec],
    out_specs=_row_spec,
    out_shape=jax.ShapeDtypeStruct((N_NODES, D), jnp.float32),
)

_fin_call = pl.pallas_call(
    _fin_body,
    grid=_grid,
    in_specs=[_a0_spec, _a1_spec, _row_spec, _dis_spec, _b_spec],
    out_specs=_row_spec,
    out_shape=jax.ShapeDtypeStruct((N_NODES, D), jnp.float32),
)


@jax.jit
def kernel(x_piece, edge_index_piece, W1, b1, W2, b2, W3, b3):
    src = edge_index_piece[0].astype(jnp.int32)
    dst = edge_index_piece[1].astype(jnp.int32)
    npad = E_PAD - N_EDGES
    pad_i = jnp.arange(npad, dtype=jnp.int32)
    dst_p = jnp.concatenate([dst, N_NODES + pad_i % (ACC_ROWS - N_NODES)])
    dst3 = dst_p.reshape(NW, NCH, CHUNK)

    b1r = b1.reshape(1, D)
    b2r = b2.reshape(1, D)
    b3r = b3.reshape(1, D)

    hist_p = _degree_kernel(dst3)
    src_b = lax.optimization_barrier(src)
    src_p = jnp.concatenate([src_b, pad_i % N_NODES])
    src3 = src_p.reshape(NW, NCH, CHUNK)
    h1_raw = _mm_call(x_piece, W1)
    h1, dis = _scale_call(h1_raw, hist_p, hist_p)
    acc = _scatter_kernel(h1, src3, dst3)
    h2 = _mid_call(acc, acc, h1, dis, b1r, W2)
    acc = _scatter_kernel(h2, src3, dst3)
    h3 = _mid_call(acc, acc, h2, dis, b2r, W3)
    acc = _scatter_kernel(h3, src3, dst3)
    return _fin_call(acc, acc, h3, dis, b3r)
